# baseline (speedup 1.0000x reference)
import numpy as np
import jax
import jax.numpy as jnp
from jax import lax
from jax.experimental import pallas as pl
from jax.experimental.pallas import tpu as pltpu
from jax.experimental.pallas import tpu_sc as plsc

_M = 262144
_ALPHA = 0.1
_REG = 0.01
_TOL = 1e-4
_CUTOFF = int(_ALPHA * _M)
_SURPLUS = 1.0 - _CUTOFF / (_ALPHA * _M)
_LOG_M = float(np.log(_M))
_INV_AM = 1.0 / (_ALPHA * _M)
_KL_HARD = _LOG_M + _CUTOFF * _INV_AM * np.log(_INV_AM) + _SURPLUS * np.log(_SURPLUS)
_LOG_INV_ALPHA = float(np.log(1.0 / _ALPHA))
_IMIN = np.int32(-(2**31))
_XMASK = np.int32(0x7FFFFFFF)

_NS = 16
_NPT = _M // _NS
_NV = _NPT // 16
_BINS = 256


def _scalar(x):
    return x if x.ndim == 0 else lax.squeeze(lax.slice(x, (0,), (1,)), (0,))


def _sc_body(v_hbm, ts_hbm, out_hbm, vv, cand, hist, stg, csbuf, cbbuf, hbuf,
             fstg, istg, fbuf, ibuf, ostg, tstg, sh_hist, sh_f, sh_i, dma_sem):
    wid = lax.axis_index("s")
    lane = lax.iota(jnp.int32, 16)
    laneb = lane * _BINS
    ones = jnp.ones((16,), jnp.int32)
    izero = jnp.zeros((16,), jnp.int32)
    fzero = jnp.zeros((16,), jnp.float32)
    iminv = jnp.full((16,), _IMIN, jnp.int32)

    in_dma = pltpu.async_copy(v_hbm.at[pl.ds(wid * _NPT, _NPT)], vv, dma_sem)

    def keyu(x):
        b = lax.bitcast_convert_type(x, jnp.int32)
        k = jnp.where(b >= 0, b, b ^ _XMASK)
        return k, k ^ _IMIN

    def _clear_hist():
        @plsc.parallel_loop(0, (16 * _BINS) // 16, unroll=8)
        def _cl(i):
            hist[pl.ds(i * 16, 16)] = izero

    def _publish_hist(lvl):
        @plsc.parallel_loop(0, _BINS // 16, unroll=2)
        def _lr(c):
            acc = izero
            for l in range(16):
                acc = acc + hist[pl.ds(l * _BINS + c * 16, 16)]
                hist[pl.ds(l * _BINS + c * 16, 16)] = izero
            stg[pl.ds(c * 16, 16)] = acc
        pltpu.sync_copy(stg, sh_hist.at[lvl, pl.ds(wid * _BINS, _BINS)])
        plsc.subcore_barrier()

    def _combine(lvl, k_rem):
        pltpu.sync_copy(sh_hist.at[lvl], hbuf)

        def _ca(c, run):
            acc = izero
            for l in range(_NS):
                acc = acc + hbuf[pl.ds(l * _BINS + c * 16, 16)]
            s = plsc.cumsum(acc) + run
            cbbuf[pl.ds(c * 16, 16)] = acc
            csbuf[pl.ds(c * 16, 16)] = s
            return _scalar(lax.slice(s, (15,), (16,)))
        T = lax.fori_loop(0, _BINS // 16, _ca, jnp.int32(0))
        thresh = T - k_rem

        def _cb(c, bstar):
            s = csbuf[pl.ds(c * 16, 16)]
            idxv = lane + c * 16
            cnd = jnp.where(s > thresh, idxv, jnp.int32(_BINS))
            return jnp.minimum(bstar, jnp.min(cnd))
        bstar = lax.fori_loop(0, _BINS // 16, _cb, jnp.int32(_BINS))

        bsplat = izero + bstar
        n_eq = _scalar(plsc.load_gather(cbbuf, [bsplat]))
        cs_at = _scalar(plsc.load_gather(csbuf, [bsplat]))
        above = T - cs_at
        return bstar, above, n_eq, k_rem - above

    _clear_hist()
    in_dma.wait()

    @plsc.parallel_loop(0, _NV, unroll=8)
    def _p1(i):
        x = vv[pl.ds(i * 16, 16)]
        _, u = keyu(x)
        b1 = lax.shift_right_logical(u, 24)
        plsc.addupdate_scatter(hist, [laneb + b1], ones)
    _publish_hist(0)
    bstar1, above1, _, k_rem = _combine(0, jnp.int32(_CUTOFF))

    @plsc.parallel_loop(0, _NV, unroll=16,
                        carry=(jnp.int32(0), fzero, iminv))
    def _p2(i, carry):
        wptr, s_ab, mb = carry
        x = vv[pl.ds(i * 16, 16)]
        k, u = keyu(x)
        b1 = lax.shift_right_logical(u, 24)
        s_ab = s_ab + jnp.where(b1 > bstar1, x, fzero)
        mb = jnp.maximum(mb, jnp.where(b1 < bstar1, k, iminv))
        keep = b1 == bstar1
        b2 = lax.shift_right_logical(u, 16) & 0xFF
        plsc.addupdate_scatter(hist, [laneb + b2], ones, mask=keep)
        plsc.store_compressed(cand.at[pl.ds(wptr, 16)], x, mask=keep)
        wptr = wptr + _scalar(plsc.all_reduce_population_count(keep))
        return (wptr, s_ab, mb)
    n_cand, s_ab, mb = _p2
    _publish_hist(1)
    bstar2, above2, _, k_rem = _combine(1, k_rem)

    nvc = ((n_cand + 63) // 64) * 4

    @plsc.parallel_loop(0, nvc, unroll=4, carry=(s_ab, mb))
    def _p3(i, carry):
        s_ab, mb = carry
        x = cand[pl.ds(i * 16, 16)]
        valid = (i * 16 + lane) < n_cand
        k, u = keyu(x)
        b2 = lax.shift_right_logical(u, 16) & 0xFF
        s_ab = s_ab + jnp.where(valid & (b2 > bstar2), x, fzero)
        mb = jnp.maximum(mb, jnp.where(valid & (b2 < bstar2), k, iminv))
        keep = valid & (b2 == bstar2)
        b3 = lax.shift_right_logical(u, 8) & 0xFF
        plsc.addupdate_scatter(hist, [laneb + b3], ones, mask=keep)
        return (s_ab, mb)
    s_ab, mb = _p3
    _publish_hist(2)
    bstar3, above3, _, k_rem = _combine(2, k_rem)

    @plsc.parallel_loop(0, nvc, unroll=4, carry=(s_ab, mb))
    def _p4(i, carry):
        s_ab, mb = carry
        x = cand[pl.ds(i * 16, 16)]
        valid = (i * 16 + lane) < n_cand
        k, u = keyu(x)
        b2 = lax.shift_right_logical(u, 16) & 0xFF
        b3 = lax.shift_right_logical(u, 8) & 0xFF
        m2 = valid & (b2 == bstar2)
        s_ab = s_ab + jnp.where(m2 & (b3 > bstar3), x, fzero)
        mb = jnp.maximum(mb, jnp.where(m2 & (b3 < bstar3), k, iminv))
        keep = m2 & (b3 == bstar3)
        b4 = u & 0xFF
        plsc.addupdate_scatter(hist, [laneb + b4], ones, mask=keep)
        return (s_ab, mb)
    s_ab, mb = _p4
    fstg[...] = s_ab
    istg[...] = mb
    pltpu.sync_copy(fstg, sh_f.at[pl.ds(wid * 16, 16)])
    pltpu.sync_copy(istg, sh_i.at[pl.ds(wid * 16, 16)])
    _publish_hist(3)
    bstar4, above4, n_eq4, k_rem = _combine(3, k_rem)

    pfx_base = (((bstar1 << 8) | bstar2) << 8 | bstar3) << 8

    def _l4(c, carry):
        s4, bel = carry
        cnt = cbbuf[pl.ds(c * 16, 16)]
        idxv = lane + c * 16
        kk = (pfx_base | idxv) ^ _IMIN
        vals = lax.bitcast_convert_type(jnp.where(kk >= 0, kk, kk ^ _XMASK), jnp.float32)
        s4 = s4 + jnp.sum(jnp.where(idxv > bstar4,
                                    vals * cnt.astype(jnp.float32), fzero))
        m = (idxv < bstar4) & (cnt > 0)
        bel = jnp.maximum(bel, jnp.max(jnp.where(m, idxv,
                                                 jnp.full((16,), -1, jnp.int32))))
        return (s4, bel)
    s4, bel4 = lax.fori_loop(0, _BINS // 16, _l4, (jnp.float32(0), jnp.int32(-1)))

    @pl.when(wid == 0)
    def _final():
        pltpu.sync_copy(sh_f, fbuf)
        pltpu.sync_copy(sh_i, ibuf)

        def _fr(t, carry):
            sg, mm = carry
            sg = sg + jnp.sum(fbuf[pl.ds(t * 16, 16)])
            mm = jnp.maximum(mm, ibuf[pl.ds(t * 16, 16)])
            return (sg, mm)
        sgt, mbg_vec = lax.fori_loop(0, _NS, _fr, (jnp.float32(0), iminv))
        mbk = jnp.max(mbg_vec)
        sgt = sgt + s4
        c_gt = above1 + above2 + above3 + above4
        c_ge = c_gt + n_eq4
        t1k = (pfx_base | bstar4) ^ _IMIN
        bel4k = jnp.where(bel4 >= 0, (pfx_base | bel4) ^ _IMIN, _IMIN)
        t2k = jnp.where(c_ge >= _CUTOFF + 1, t1k, jnp.maximum(mbk, bel4k))

        def unkey_splat(kscalar):
            kv = izero + kscalar
            return lax.bitcast_convert_type(jnp.where(kv >= 0, kv, kv ^ _XMASK), jnp.float32)
        t1v = unkey_splat(t1k)
        t2v = unkey_splat(t2k)
        s_top = sgt + (jnp.float32(_CUTOFF) - c_gt.astype(jnp.float32)) * t1v
        hard = s_top * _INV_AM + _SURPLUS * t2v - _REG * _KL_HARD
        pltpu.sync_copy(ts_hbm.at[pl.ds(0, 16)], tstg)
        tsv = tstg[...]
        target = _scalar(lax.slice(tsv, (0,), (1,)))
        soft = _scalar(lax.slice(tsv, (1,), (2,)))
        ostg[...] = jnp.where(jnp.abs(target) <= _TOL, fzero + soft, hard)
        pltpu.sync_copy(ostg, out_hbm)


def _tc_body(v_ref, out_ref):
    v = v_ref[...]
    vmax = jnp.max(v)
    e = jnp.exp((v - vmax) * (1.0 / _REG))
    S = jnp.sum(e)
    w = jnp.minimum(e * (_M / S), 1.0 / _ALPHA)
    target = 1.0 - jnp.sum(w) * (1.0 / _M)
    dot_soft = jnp.sum(w * v) * (1.0 / _M)
    x = (v - vmax) * (1.0 / _REG) - jnp.log(S) + _LOG_M
    ent = jnp.sum(w * (jnp.minimum(x, _LOG_INV_ALPHA) - _LOG_M)) * (1.0 / _M)
    soft_val = dot_soft - _REG * (_LOG_M + ent)
    il = lax.broadcasted_iota(jnp.int32, (1, 128), 1)
    out_ref[...] = jnp.where(il == 0, target, soft_val)


_SC_SCRATCH = [
    pltpu.VMEM((_NPT,), jnp.float32),
    pltpu.VMEM((_NPT + 128,), jnp.float32),
    pltpu.VMEM((16 * _BINS,), jnp.int32),
    pltpu.VMEM((_BINS,), jnp.int32),
    pltpu.VMEM((_BINS,), jnp.int32),
    pltpu.VMEM((_BINS,), jnp.int32),
    pltpu.VMEM((_NS * _BINS,), jnp.int32),
    pltpu.VMEM((16,), jnp.float32),
    pltpu.VMEM((16,), jnp.int32),
    pltpu.VMEM((_NS * 16,), jnp.float32),
    pltpu.VMEM((_NS * 16,), jnp.int32),
    pltpu.VMEM((16,), jnp.float32),
    pltpu.VMEM((16,), jnp.float32),
    pltpu.VMEM_SHARED((4, _NS * _BINS), jnp.int32),
    pltpu.VMEM_SHARED((_NS * 16,), jnp.float32),
    pltpu.VMEM_SHARED((_NS * 16,), jnp.int32),
    pltpu.SemaphoreType.DMA,
]


def kernel(v):
    sc_fn = pl.kernel(
        _sc_body,
        out_type=jax.ShapeDtypeStruct((16,), jnp.float32),
        mesh=plsc.VectorSubcoreMesh(core_axis_name="c", subcore_axis_name="s",
                                    num_cores=1),
        scratch_types=_SC_SCRATCH,
        compiler_params=pltpu.CompilerParams(needs_layout_passes=False),
    )
    ts = pl.pallas_call(
        _tc_body,
        out_shape=jax.ShapeDtypeStruct((1, 128), jnp.float32),
    )(v.reshape(2048, 128))
    out16 = sc_fn(v, ts.reshape(128))
    return out16[0]


# speedup vs baseline: 1.0092x; 1.0092x over previous
import numpy as np
import jax
import jax.numpy as jnp
from jax import lax
from jax.experimental import pallas as pl
from jax.experimental.pallas import tpu as pltpu
from jax.experimental.pallas import tpu_sc as plsc

_M = 262144
_ALPHA = 0.1
_REG = 0.01
_TOL = 1e-4
_CUTOFF = int(_ALPHA * _M)
_SURPLUS = 1.0 - _CUTOFF / (_ALPHA * _M)
_LOG_M = float(np.log(_M))
_INV_AM = 1.0 / (_ALPHA * _M)
_KL_HARD = _LOG_M + _CUTOFF * _INV_AM * np.log(_INV_AM) + _SURPLUS * np.log(_SURPLUS)
_LOG_INV_ALPHA = float(np.log(1.0 / _ALPHA))
_IMIN = np.int32(-(2**31))
_XMASK = np.int32(0x7FFFFFFF)

_NS = 16
_NPT = _M // _NS
_NV = _NPT // 16
_BINS = 256


def _scalar(x):
    return x if x.ndim == 0 else lax.squeeze(lax.slice(x, (0,), (1,)), (0,))


def _sc_body(v_hbm, ts_hbm, out_hbm, vv, cand, hist, stg, csbuf, cbbuf, hbuf,
             fstg, istg, fbuf, ibuf, ostg, tstg, sh_hist, sh_f, sh_i, dma_sem):
    wid = lax.axis_index("s")
    lane = lax.iota(jnp.int32, 16)
    laneb = lane * _BINS
    ones = jnp.ones((16,), jnp.int32)
    izero = jnp.zeros((16,), jnp.int32)
    fzero = jnp.zeros((16,), jnp.float32)
    iminv = jnp.full((16,), _IMIN, jnp.int32)

    in_dma = pltpu.async_copy(v_hbm.at[pl.ds(wid * _NPT, _NPT)], vv, dma_sem)

    def keyu(x):
        b = lax.bitcast_convert_type(x, jnp.int32)
        k = jnp.where(b >= 0, b, b ^ _XMASK)
        return k, k ^ _IMIN

    def _clear_hist():
        @plsc.parallel_loop(0, (16 * _BINS) // 16, unroll=8)
        def _cl(i):
            hist[pl.ds(i * 16, 16)] = izero

    def _publish_hist(lvl):
        @plsc.parallel_loop(0, _BINS // 16, unroll=2)
        def _lr(c):
            acc = izero
            for l in range(16):
                acc = acc + hist[pl.ds(l * _BINS + c * 16, 16)]
                hist[pl.ds(l * _BINS + c * 16, 16)] = izero
            stg[pl.ds(c * 16, 16)] = acc
        pltpu.sync_copy(stg, sh_hist.at[lvl, pl.ds(wid * _BINS, _BINS)])
        plsc.subcore_barrier()

    def _combine(lvl, k_rem):
        pltpu.sync_copy(sh_hist.at[lvl], hbuf)

        def _ca(c, run):
            acc = izero
            for l in range(_NS):
                acc = acc + hbuf[pl.ds(l * _BINS + c * 16, 16)]
            s = plsc.cumsum(acc) + run
            cbbuf[pl.ds(c * 16, 16)] = acc
            csbuf[pl.ds(c * 16, 16)] = s
            return _scalar(lax.slice(s, (15,), (16,)))
        T = lax.fori_loop(0, _BINS // 16, _ca, jnp.int32(0))
        thresh = T - k_rem

        def _cb(c, bstar):
            s = csbuf[pl.ds(c * 16, 16)]
            idxv = lane + c * 16
            cnd = jnp.where(s > thresh, idxv, jnp.int32(_BINS))
            return jnp.minimum(bstar, jnp.min(cnd))
        bstar = lax.fori_loop(0, _BINS // 16, _cb, jnp.int32(_BINS))

        bsplat = izero + bstar
        n_eq = _scalar(plsc.load_gather(cbbuf, [bsplat]))
        cs_at = _scalar(plsc.load_gather(csbuf, [bsplat]))
        above = T - cs_at
        return bstar, above, n_eq, k_rem - above

    _clear_hist()
    in_dma.wait()

    @plsc.parallel_loop(0, _NV, unroll=8)
    def _p1(i):
        x = vv[pl.ds(i * 16, 16)]
        _, u = keyu(x)
        b1 = lax.shift_right_logical(u, 24)
        plsc.addupdate_scatter(hist, [laneb + b1], ones)
    _publish_hist(0)
    bstar1, above1, _, k_rem = _combine(0, jnp.int32(_CUTOFF))

    @plsc.parallel_loop(0, _NV, unroll=8,
                        carry=(jnp.int32(0), fzero, iminv))
    def _p2(i, carry):
        wptr, s_ab, mb = carry
        x = vv[pl.ds(i * 16, 16)]
        k, u = keyu(x)
        b1 = lax.shift_right_logical(u, 24)
        s_ab = s_ab + jnp.where(b1 > bstar1, x, fzero)
        mb = jnp.maximum(mb, jnp.where(b1 < bstar1, k, iminv))
        keep = b1 == bstar1
        b2 = lax.shift_right_logical(u, 16) & 0xFF
        plsc.addupdate_scatter(hist, [laneb + b2], ones, mask=keep)
        plsc.store_compressed(cand.at[pl.ds(wptr, 16)], x, mask=keep)
        wptr = wptr + _scalar(plsc.all_reduce_population_count(keep))
        return (wptr, s_ab, mb)
    n_cand, s_ab, mb = _p2
    _publish_hist(1)
    bstar2, above2, _, k_rem = _combine(1, k_rem)

    nvc = ((n_cand + 63) // 64) * 4

    @plsc.parallel_loop(0, nvc, unroll=4, carry=(s_ab, mb))
    def _p3(i, carry):
        s_ab, mb = carry
        x = cand[pl.ds(i * 16, 16)]
        valid = (i * 16 + lane) < n_cand
        k, u = keyu(x)
        b2 = lax.shift_right_logical(u, 16) & 0xFF
        s_ab = s_ab + jnp.where(valid & (b2 > bstar2), x, fzero)
        mb = jnp.maximum(mb, jnp.where(valid & (b2 < bstar2), k, iminv))
        keep = valid & (b2 == bstar2)
        b3 = lax.shift_right_logical(u, 8) & 0xFF
        plsc.addupdate_scatter(hist, [laneb + b3], ones, mask=keep)
        return (s_ab, mb)
    s_ab, mb = _p3
    _publish_hist(2)
    bstar3, above3, _, k_rem = _combine(2, k_rem)

    @plsc.parallel_loop(0, nvc, unroll=4, carry=(s_ab, mb))
    def _p4(i, carry):
        s_ab, mb = carry
        x = cand[pl.ds(i * 16, 16)]
        valid = (i * 16 + lane) < n_cand
        k, u = keyu(x)
        b2 = lax.shift_right_logical(u, 16) & 0xFF
        b3 = lax.shift_right_logical(u, 8) & 0xFF
        m2 = valid & (b2 == bstar2)
        s_ab = s_ab + jnp.where(m2 & (b3 > bstar3), x, fzero)
        mb = jnp.maximum(mb, jnp.where(m2 & (b3 < bstar3), k, iminv))
        keep = m2 & (b3 == bstar3)
        b4 = u & 0xFF
        plsc.addupdate_scatter(hist, [laneb + b4], ones, mask=keep)
        return (s_ab, mb)
    s_ab, mb = _p4
    fstg[...] = s_ab
    istg[...] = mb
    pltpu.sync_copy(fstg, sh_f.at[pl.ds(wid * 16, 16)])
    pltpu.sync_copy(istg, sh_i.at[pl.ds(wid * 16, 16)])
    _publish_hist(3)
    bstar4, above4, n_eq4, k_rem = _combine(3, k_rem)

    pfx_base = (((bstar1 << 8) | bstar2) << 8 | bstar3) << 8

    def _l4(c, carry):
        s4, bel = carry
        cnt = cbbuf[pl.ds(c * 16, 16)]
        idxv = lane + c * 16
        kk = (pfx_base | idxv) ^ _IMIN
        vals = lax.bitcast_convert_type(jnp.where(kk >= 0, kk, kk ^ _XMASK), jnp.float32)
        s4 = s4 + jnp.sum(jnp.where(idxv > bstar4,
                                    vals * cnt.astype(jnp.float32), fzero))
        m = (idxv < bstar4) & (cnt > 0)
        bel = jnp.maximum(bel, jnp.max(jnp.where(m, idxv,
                                                 jnp.full((16,), -1, jnp.int32))))
        return (s4, bel)
    s4, bel4 = lax.fori_loop(0, _BINS // 16, _l4, (jnp.float32(0), jnp.int32(-1)))

    @pl.when(wid == 0)
    def _final():
        pltpu.sync_copy(sh_f, fbuf)
        pltpu.sync_copy(sh_i, ibuf)

        def _fr(t, carry):
            sg, mm = carry
            sg = sg + jnp.sum(fbuf[pl.ds(t * 16, 16)])
            mm = jnp.maximum(mm, ibuf[pl.ds(t * 16, 16)])
            return (sg, mm)
        sgt, mbg_vec = lax.fori_loop(0, _NS, _fr, (jnp.float32(0), iminv))
        mbk = jnp.max(mbg_vec)
        sgt = sgt + s4
        c_gt = above1 + above2 + above3 + above4
        c_ge = c_gt + n_eq4
        t1k = (pfx_base | bstar4) ^ _IMIN
        bel4k = jnp.where(bel4 >= 0, (pfx_base | bel4) ^ _IMIN, _IMIN)
        t2k = jnp.where(c_ge >= _CUTOFF + 1, t1k, jnp.maximum(mbk, bel4k))

        def unkey_splat(kscalar):
            kv = izero + kscalar
            return lax.bitcast_convert_type(jnp.where(kv >= 0, kv, kv ^ _XMASK), jnp.float32)
        t1v = unkey_splat(t1k)
        t2v = unkey_splat(t2k)
        s_top = sgt + (jnp.float32(_CUTOFF) - c_gt.astype(jnp.float32)) * t1v
        hard = s_top * _INV_AM + _SURPLUS * t2v - _REG * _KL_HARD
        pltpu.sync_copy(ts_hbm.at[pl.ds(0, 16)], tstg)
        tsv = tstg[...]
        target = _scalar(lax.slice(tsv, (0,), (1,)))
        soft = _scalar(lax.slice(tsv, (1,), (2,)))
        ostg[...] = jnp.where(jnp.abs(target) <= _TOL, fzero + soft, hard)
        pltpu.sync_copy(ostg, out_hbm)


def _tc_body(v_ref, out_ref):
    v = v_ref[...]
    vmax = jnp.max(v)
    e = jnp.exp((v - vmax) * (1.0 / _REG))
    S = jnp.sum(e)
    w = jnp.minimum(e * (_M / S), 1.0 / _ALPHA)
    target = 1.0 - jnp.sum(w) * (1.0 / _M)
    dot_soft = jnp.sum(w * v) * (1.0 / _M)
    x = (v - vmax) * (1.0 / _REG) - jnp.log(S) + _LOG_M
    ent = jnp.sum(w * (jnp.minimum(x, _LOG_INV_ALPHA) - _LOG_M)) * (1.0 / _M)
    soft_val = dot_soft - _REG * (_LOG_M + ent)
    il = lax.broadcasted_iota(jnp.int32, (1, 128), 1)
    out_ref[...] = jnp.where(il == 0, target, soft_val)


_SC_SCRATCH = [
    pltpu.VMEM((_NPT,), jnp.float32),
    pltpu.VMEM((_NPT + 128,), jnp.float32),
    pltpu.VMEM((16 * _BINS,), jnp.int32),
    pltpu.VMEM((_BINS,), jnp.int32),
    pltpu.VMEM((_BINS,), jnp.int32),
    pltpu.VMEM((_BINS,), jnp.int32),
    pltpu.VMEM((_NS * _BINS,), jnp.int32),
    pltpu.VMEM((16,), jnp.float32),
    pltpu.VMEM((16,), jnp.int32),
    pltpu.VMEM((_NS * 16,), jnp.float32),
    pltpu.VMEM((_NS * 16,), jnp.int32),
    pltpu.VMEM((16,), jnp.float32),
    pltpu.VMEM((16,), jnp.float32),
    pltpu.VMEM_SHARED((4, _NS * _BINS), jnp.int32),
    pltpu.VMEM_SHARED((_NS * 16,), jnp.float32),
    pltpu.VMEM_SHARED((_NS * 16,), jnp.int32),
    pltpu.SemaphoreType.DMA,
]


def kernel(v):
    sc_fn = pl.kernel(
        _sc_body,
        out_type=jax.ShapeDtypeStruct((16,), jnp.float32),
        mesh=plsc.VectorSubcoreMesh(core_axis_name="c", subcore_axis_name="s",
                                    num_cores=1),
        scratch_types=_SC_SCRATCH,
        compiler_params=pltpu.CompilerParams(needs_layout_passes=False),
    )
    ts = pl.pallas_call(
        _tc_body,
        out_shape=jax.ShapeDtypeStruct((1, 128), jnp.float32),
    )(v.reshape(2048, 128))
    out16 = sc_fn(v, ts.reshape(128))
    return out16[0]
